# speedup vs baseline: 1.9938x; 1.9938x over previous
import functools

import jax
import jax.numpy as jnp
from jax import lax
from jax.experimental import pallas as pl
from jax.experimental.pallas import tpu as pltpu
from jax.experimental.pallas import tpu_sc as plsc


def _argmin_body(k_total, z_ref, cb_ref, idx_ref, enorm_ref):
    @pl.when(pl.program_id(0) == 0)
    def _():
        cb0 = cb_ref[...]
        enorm_ref[...] = jnp.sum(cb0 * cb0, axis=1, keepdims=True).T

    z = z_ref[...]
    zn = jnp.sum(z * z, axis=1, keepdims=True)
    dot = lax.dot_general(z, cb_ref[...], (((1,), (1,)), ((), ())),
                          preferred_element_type=jnp.float32)
    dist = (zn + enorm_ref[...]) - 2.0 * dot
    m = jnp.min(dist, axis=1, keepdims=True)
    ii = lax.broadcasted_iota(jnp.int32, dist.shape, 1)
    idx_ref[...] = jnp.min(jnp.where(dist == m, ii, k_total), axis=1)


def _hard_indices(z_flat, codebook, tb=256):
    n, d = z_flat.shape
    k = codebook.shape[0]
    grid = n // tb
    return pl.pallas_call(
        functools.partial(_argmin_body, k),
        grid=(grid,),
        in_specs=[
            pl.BlockSpec((tb, d), lambda i: (i, 0)),
            pl.BlockSpec((k, d), lambda i: (0, 0)),
        ],
        out_specs=pl.BlockSpec((tb,), lambda i: (i,)),
        out_shape=jax.ShapeDtypeStruct((n,), jnp.int32),
        scratch_shapes=[pltpu.VMEM((1, k), jnp.float32)],
    )(z_flat, codebook)


def _gather_rows(idx, table):
    n = idx.shape[0]
    k, d = table.shape
    info = plsc.get_sparse_core_info()
    nw = info.num_cores * info.num_subcores
    per_w = n // nw
    chunk = 128
    n_chunks = per_w // chunk
    mesh = plsc.VectorSubcoreMesh(core_axis_name="c", subcore_axis_name="s")

    @functools.partial(
        pl.kernel,
        out_type=jax.ShapeDtypeStruct((n, d), jnp.float32),
        mesh=mesh,
        scratch_types=[
            pltpu.VMEM((chunk,), jnp.int32),
            pltpu.VMEM((chunk, d), jnp.float32),
            pltpu.SemaphoreType.DMA,
        ],
    )
    def gather_kernel(idx_hbm, table_hbm, out_hbm, idx_v, rows_v, sem):
        wid = lax.axis_index("s") * info.num_cores + lax.axis_index("c")
        base_w = wid * per_w
        for c in range(n_chunks):
            base = base_w + c * chunk
            pltpu.sync_copy(idx_hbm.at[pl.ds(base, chunk)], idx_v)
            pltpu.async_copy(table_hbm.at[idx_v], rows_v, sem).wait()
            pltpu.sync_copy(rows_v, out_hbm.at[pl.ds(base, chunk)])

    return gather_kernel(idx, table)


def _finalize_body(nsteps, total, k_total, z_ref, e_ref, u_ref,
                   zq_ref, com_ref, per_ref, ent_ref, acc_ref):
    i = pl.program_id(0)

    @pl.when(i == 0)
    def _():
        acc_ref[0, 0] = 0.0

    z = z_ref[...]
    e = e_ref[...]
    zq = z + (e - z)
    zq_ref[...] = zq
    dd = zq - z
    acc_ref[0, 0] += jnp.sum(dd * dd)

    @pl.when(i == nsteps - 1)
    def _():
        com_ref[0, 0] = acc_ref[0, 0] / total
        u = u_ref[...]
        s = jnp.sum(u)
        p = jnp.where(s > 0, u / (s + 1e-10),
                      jnp.full_like(u, 1.0 / k_total))
        ent = -jnp.sum(p * jnp.log(p + 1e-10))
        ent_ref[0, 0] = ent
        per_ref[0, 0] = jnp.exp(ent)


def _finalize(z_flat, emb_flat, usage, tb=512):
    n, d = z_flat.shape
    k = usage.shape[0]
    grid = n // tb
    usage2 = usage.reshape(1, k)
    scalar = jax.ShapeDtypeStruct((1, 1), jnp.float32)
    zq, com, per, ent = pl.pallas_call(
        functools.partial(_finalize_body, grid, float(n * d), k),
        grid=(grid,),
        in_specs=[
            pl.BlockSpec((tb, d), lambda i: (i, 0)),
            pl.BlockSpec((tb, d), lambda i: (i, 0)),
            pl.BlockSpec((1, k), lambda i: (0, 0)),
        ],
        out_specs=[
            pl.BlockSpec((tb, d), lambda i: (i, 0)),
            pl.BlockSpec(memory_space=pltpu.SMEM),
            pl.BlockSpec(memory_space=pltpu.SMEM),
            pl.BlockSpec(memory_space=pltpu.SMEM),
        ],
        out_shape=[
            jax.ShapeDtypeStruct((n, d), jnp.float32),
            scalar, scalar, scalar,
        ],
        scratch_shapes=[pltpu.SMEM((1, 1), jnp.float32)],
    )(z_flat, emb_flat, usage2)
    return zq, com[0, 0], per[0, 0], ent[0, 0]


def kernel(z, codebook, codebook_usage, training):
    b, t, d = z.shape
    z_flat = z.reshape(-1, d)
    idx = _hard_indices(z_flat, codebook)
    emb_flat = _gather_rows(idx, codebook)
    zq_flat, commitment, perplexity, entropy = _finalize(
        z_flat, emb_flat, codebook_usage)
    return (zq_flat.reshape(b, t, d), emb_flat.reshape(b, t, d),
            idx.reshape(b, t), commitment, perplexity, entropy)
